# baseline (speedup 1.0000x reference)
import functools
import jax
import jax.numpy as jnp
from jax import lax
from jax.experimental import pallas as pl
from jax.experimental.pallas import tpu as pltpu
from jax.experimental.pallas import tpu_sc as plsc

_N = 100000
_D = 128
_NC = 2
_NS = 16
_NW = _NC * _NS
_CH = 200
_G = _N // _CH
_VECS = _CH * _D // 16

_mesh = plsc.VectorSubcoreMesh(core_axis_name="c", subcore_axis_name="s")


@functools.partial(
    pl.kernel,
    mesh=_mesh,
    out_type=jax.ShapeDtypeStruct((_N, _D), jnp.float32),
    scratch_types=[
        pltpu.VMEM((_CH, _D), jnp.float32),
        pltpu.VMEM((_CH, _D), jnp.float32),
    ],
)
def _sc_sigmoid(p_hbm, z_hbm, inb, outb):
    wid = lax.axis_index("s") * _NC + lax.axis_index("c")
    trip = (_G - wid + _NW - 1) // _NW

    def chunk_body(t, _):
        g = wid + t * _NW
        row0 = pl.multiple_of(g * _CH, 8)
        pltpu.sync_copy(p_hbm.at[pl.ds(row0, _CH)], inb)

        def vec_body(j, _):
            r = j // 8
            k = (j % 8) * 16
            x = inb[r, pl.ds(k, 16)]
            outb[r, pl.ds(k, 16)] = 1.0 / (1.0 + jnp.exp(-x))
            return 0

        lax.fori_loop(0, _VECS, vec_body, 0)
        pltpu.sync_copy(outb, z_hbm.at[pl.ds(row0, _CH)])
        return 0

    lax.fori_loop(0, trip, chunk_body, 0)


def kernel(P, test):
    return _sc_sigmoid(P)


# speedup vs baseline: 4.5246x; 4.5246x over previous
import jax
import jax.numpy as jnp
from jax.experimental import pallas as pl

_N = 100000
_D = 128
_BLOCK = 25000


def _sigmoid_block(p_ref, z_ref):
    z_ref[...] = jax.nn.sigmoid(p_ref[...])


def kernel(P, test):
    return pl.pallas_call(
        _sigmoid_block,
        grid=(_N // _BLOCK,),
        in_specs=[pl.BlockSpec((_BLOCK, _D), lambda i: (i, 0))],
        out_specs=pl.BlockSpec((_BLOCK, _D), lambda i: (i, 0)),
        out_shape=jax.ShapeDtypeStruct((_N, _D), jnp.float32),
    )(P)


# speedup vs baseline: 4.8737x; 1.0772x over previous
import jax
import jax.numpy as jnp
from jax.experimental import pallas as pl

_N = 100000
_D = 128
_BLOCK = 25000


def _sigmoid_block(p_ref, z_ref):
    z_ref[...] = 0.5 * jnp.tanh(p_ref[...] * 0.5) + 0.5


def kernel(P, test):
    return pl.pallas_call(
        _sigmoid_block,
        grid=(_N // _BLOCK,),
        in_specs=[pl.BlockSpec((_BLOCK, _D), lambda i: (i, 0))],
        out_specs=pl.BlockSpec((_BLOCK, _D), lambda i: (i, 0)),
        out_shape=jax.ShapeDtypeStruct((_N, _D), jnp.float32),
    )(P)
